# baseline (speedup 1.0000x reference)
import functools

import jax
import jax.numpy as jnp
from jax import lax
from jax.experimental import pallas as pl
from jax.experimental.pallas import tpu as pltpu
from jax.experimental.pallas import tpu_sc as plsc

_THRESH = 0.001
_ROWS, _COLS = 128, 32768
_NC, _NS, _L = 2, 16, 16
_NW = _NC * _NS
_RPW = _ROWS // _NW

_mesh = plsc.VectorSubcoreMesh(core_axis_name="c", subcore_axis_name="s")


@functools.partial(
    pl.kernel,
    out_type=(
        jax.ShapeDtypeStruct((_ROWS, _COLS), jnp.float32),
        jax.ShapeDtypeStruct((_NW, _RPW, _L), jnp.float32),
    ),
    mesh=_mesh,
    compiler_params=pltpu.CompilerParams(needs_layout_passes=False),
    scratch_types=(
        pltpu.VMEM((_COLS,), jnp.float32),
        pltpu.VMEM((_COLS,), jnp.float32),
        pltpu.VMEM((_RPW, _L), jnp.float32),
    ),
)
def _sc_mask_count(x_hbm, mask_hbm, cnt_hbm, x_v, m_v, c_v):
    wid = lax.axis_index("s") * _NC + lax.axis_index("c")
    for r in range(_RPW):
        row = wid * _RPW + r
        pltpu.sync_copy(x_hbm.at[row], x_v)

        def body(i, acc):
            v = x_v[pl.ds(i * _L, _L)]
            m = jnp.abs(v) > _THRESH
            m_v[pl.ds(i * _L, _L)] = jnp.where(m, 1.0, 0.0)
            return acc + plsc.all_reduce_population_count(m)

        acc = lax.fori_loop(0, _COLS // _L, body, jnp.zeros((_L,), jnp.int32))
        c_v[r] = acc.astype(jnp.float32)
        pltpu.sync_copy(m_v, mask_hbm.at[row])
    pltpu.sync_copy(c_v, cnt_hbm.at[wid])


def kernel(x):
    mask, cnt = _sc_mask_count(x)
    sparsity = cnt[:, :, 0].reshape(_ROWS)
    return (x, sparsity, mask)


# speedup vs baseline: 1.3173x; 1.3173x over previous
import functools

import jax
import jax.numpy as jnp
from jax import lax
from jax.experimental import pallas as pl
from jax.experimental.pallas import tpu as pltpu
from jax.experimental.pallas import tpu_sc as plsc

_THRESH = 0.001
_ROWS, _COLS = 128, 32768
_NC, _NS, _L = 2, 16, 16
_NW = _NC * _NS
_RPW = _ROWS // _NW

_mesh = plsc.VectorSubcoreMesh(core_axis_name="c", subcore_axis_name="s")


@functools.partial(
    pl.kernel,
    out_type=(
        jax.ShapeDtypeStruct((_ROWS, _COLS), jnp.float32),
        jax.ShapeDtypeStruct((_NW, _RPW, _L), jnp.float32),
    ),
    mesh=_mesh,
    compiler_params=pltpu.CompilerParams(needs_layout_passes=False),
    scratch_types=(
        pltpu.VMEM((_COLS,), jnp.float32),
        pltpu.VMEM((_COLS,), jnp.float32),
        pltpu.VMEM((_RPW, _L), jnp.float32),
    ),
)
def _sc_mask_count(x_hbm, mask_hbm, cnt_hbm, x_v, m_v, c_v):
    wid = lax.axis_index("s") * _NC + lax.axis_index("c")
    for r in range(_RPW):
        row = wid * _RPW + r
        pltpu.sync_copy(x_hbm.at[row], x_v)

        def body(i, acc):
            v = x_v[pl.ds(i * _L, _L)]
            mv = jnp.where(jnp.abs(v) > _THRESH, 1.0, 0.0)
            m_v[pl.ds(i * _L, _L)] = mv
            return acc + mv

        acc = plsc.parallel_loop(
            0, _COLS // _L, 1, unroll=8,
            carry=jnp.zeros((_L,), jnp.float32))(body)
        c_v[r] = jnp.full((_L,), jnp.sum(acc), jnp.float32)
        pltpu.sync_copy(m_v, mask_hbm.at[row])
    pltpu.sync_copy(c_v, cnt_hbm.at[wid])


def kernel(x):
    mask, cnt = _sc_mask_count(x)
    sparsity = cnt[:, :, 0].reshape(_ROWS)
    return (x, sparsity, mask)


# speedup vs baseline: 1.3821x; 1.0492x over previous
import functools

import jax
import jax.numpy as jnp
from jax import lax
from jax.experimental import pallas as pl
from jax.experimental.pallas import tpu as pltpu
from jax.experimental.pallas import tpu_sc as plsc

_THRESH = 0.001
_ROWS, _COLS = 128, 32768
_NC, _NS, _L = 2, 16, 16
_NW = _NC * _NS
_RPW = _ROWS // _NW
_CHUNK = _COLS // 2
_NCHUNK = _RPW * 2
_VPC = _CHUNK // _L

_mesh = plsc.VectorSubcoreMesh(core_axis_name="c", subcore_axis_name="s")


@functools.partial(
    pl.kernel,
    out_type=(
        jax.ShapeDtypeStruct((_ROWS, _COLS), jnp.float32),
        jax.ShapeDtypeStruct((_NW, _RPW, _L), jnp.float32),
    ),
    mesh=_mesh,
    compiler_params=pltpu.CompilerParams(needs_layout_passes=False),
    scratch_types=(
        pltpu.VMEM((2, _CHUNK), jnp.float32),
        pltpu.VMEM((2, _CHUNK), jnp.float32),
        pltpu.VMEM((_RPW, _L), jnp.float32),
        pltpu.SemaphoreType.DMA,
        pltpu.SemaphoreType.DMA,
        pltpu.SemaphoreType.DMA,
        pltpu.SemaphoreType.DMA,
    ),
)
def _sc_mask_count(x_hbm, mask_hbm, cnt_hbm, x_v, m_v, c_v,
                   ls0, ls1, ss0, ss1):
    wid = lax.axis_index("s") * _NC + lax.axis_index("c")
    lsem = (ls0, ls1)
    ssem = (ss0, ss1)

    def src(k):
        row = wid * _RPW + k // 2
        return x_hbm.at[row, pl.ds((k % 2) * _CHUNK, _CHUNK)]

    def dst(k):
        row = wid * _RPW + k // 2
        return mask_hbm.at[row, pl.ds((k % 2) * _CHUNK, _CHUNK)]

    loads = [None, None]
    stores = [None, None]
    loads[0] = pltpu.async_copy(src(0), x_v.at[0], lsem[0])
    acc = jnp.zeros((_L,), jnp.float32)
    for k in range(_NCHUNK):
        s = k % 2
        if k + 1 < _NCHUNK:
            loads[(k + 1) % 2] = pltpu.async_copy(
                src(k + 1), x_v.at[(k + 1) % 2], lsem[(k + 1) % 2])
        loads[s].wait()
        if stores[s] is not None:
            stores[s].wait()

        def body(i, a):
            v = x_v[s, pl.ds(i * _L, _L)]
            mv = jnp.where(jnp.abs(v) > _THRESH, 1.0, 0.0)
            m_v[s, pl.ds(i * _L, _L)] = mv
            return a + mv

        acc = plsc.parallel_loop(0, _VPC, 1, unroll=8, carry=acc)(body)
        stores[s] = pltpu.async_copy(m_v.at[s], dst(k), ssem[s])
        if k % 2 == 1:
            c_v[k // 2] = jnp.full((_L,), jnp.sum(acc), jnp.float32)
            acc = jnp.zeros((_L,), jnp.float32)
    stores[0].wait()
    stores[1].wait()
    pltpu.sync_copy(c_v, cnt_hbm.at[wid])


def kernel(x):
    mask, cnt = _sc_mask_count(x)
    sparsity = cnt[:, :, 0].reshape(_ROWS)
    return (x, sparsity, mask)


# speedup vs baseline: 1.7944x; 1.2983x over previous
import functools

import jax
import jax.numpy as jnp
from jax import lax
from jax.experimental import pallas as pl
from jax.experimental.pallas import tpu as pltpu
from jax.experimental.pallas import tpu_sc as plsc

_THRESH = 0.001
_ROWS, _COLS = 128, 32768
_NC, _NS, _L = 2, 16, 16
_NW = _NC * _NS
_RPW = _ROWS // _NW

_mesh = plsc.VectorSubcoreMesh(core_axis_name="c", subcore_axis_name="s")


@functools.partial(
    pl.kernel,
    out_type=jax.ShapeDtypeStruct((_NW, _RPW, _L), jnp.float32),
    mesh=_mesh,
    compiler_params=pltpu.CompilerParams(needs_layout_passes=False),
    scratch_types=(
        pltpu.VMEM((2, _COLS), jnp.float32),
        pltpu.VMEM((_RPW, _L), jnp.float32),
        pltpu.SemaphoreType.DMA,
        pltpu.SemaphoreType.DMA,
    ),
)
def _sc_count(x_hbm, cnt_hbm, x_v, c_v, ls0, ls1):
    wid = lax.axis_index("s") * _NC + lax.axis_index("c")
    lsem = (ls0, ls1)
    loads = [None, None]
    loads[0] = pltpu.async_copy(x_hbm.at[wid * _RPW], x_v.at[0], lsem[0])
    for r in range(_RPW):
        s = r % 2
        if r + 1 < _RPW:
            loads[(r + 1) % 2] = pltpu.async_copy(
                x_hbm.at[wid * _RPW + r + 1], x_v.at[(r + 1) % 2],
                lsem[(r + 1) % 2])
        loads[s].wait()

        def body(i, a):
            v = x_v[s, pl.ds(i * _L, _L)]
            return a + jnp.where(jnp.abs(v) > _THRESH, 1.0, 0.0)

        acc = plsc.parallel_loop(
            0, _COLS // _L, 1, unroll=8,
            carry=jnp.zeros((_L,), jnp.float32))(body)
        c_v[r] = jnp.full((_L,), jnp.sum(acc), jnp.float32)
    pltpu.sync_copy(c_v, cnt_hbm.at[wid])


_BR = 8


def _tc_body(x_ref, copy_ref, mask_ref):
    v = x_ref[...]
    copy_ref[...] = v
    mask_ref[...] = jnp.where(jnp.abs(v) > _THRESH, 1.0, 0.0)


_tc_mask_copy = pl.pallas_call(
    _tc_body,
    grid=(_ROWS // _BR,),
    in_specs=[pl.BlockSpec((_BR, _COLS), lambda i: (i, 0))],
    out_specs=[
        pl.BlockSpec((_BR, _COLS), lambda i: (i, 0)),
        pl.BlockSpec((_BR, _COLS), lambda i: (i, 0)),
    ],
    out_shape=[
        jax.ShapeDtypeStruct((_ROWS, _COLS), jnp.float32),
        jax.ShapeDtypeStruct((_ROWS, _COLS), jnp.float32),
    ],
)


def kernel(x):
    cnt = _sc_count(x)
    x_sparse, mask = _tc_mask_copy(x)
    sparsity = cnt[:, :, 0].reshape(_ROWS)
    return (x_sparse, sparsity, mask)
